# baseline (speedup 1.0000x reference)
import functools
import jax
import jax.numpy as jnp
from jax import lax
from jax.experimental import pallas as pl
from jax.experimental.pallas import tpu as pltpu
from jax.experimental.pallas import tpu_sc as plsc

E = 8
K = 2
H = 1024
I = 2048
T = 2048

BT = 256
P = ((K * T + E * (BT - 1)) + BT - 1) // BT * BT
NTILES = P // BT
NEG_INF = -1e30

NW = 32
TPW = T // NW


def _route_body(l3_ref, dp0_ref, dp1_ref, wa_ref, wb_ref, eot_ref,
                flag_ref, slot_ref):
    sl = [l3_ref[e] for e in range(E)]
    m1 = sl[0]
    for e in range(1, E):
        m1 = jnp.maximum(m1, sl[e])
    idx1 = jnp.zeros((16, 128), jnp.int32)
    for e in range(E - 1, -1, -1):
        idx1 = jnp.where(sl[e] == m1, e, idx1)
    m2 = jnp.full((16, 128), NEG_INF, jnp.float32)
    for e in range(E):
        m2 = jnp.maximum(m2, jnp.where(idx1 == e, NEG_INF, sl[e]))
    idx2 = jnp.zeros((16, 128), jnp.int32)
    for e in range(E - 1, -1, -1):
        masked = jnp.where(idx1 == e, NEG_INF, sl[e])
        idx2 = jnp.where(masked == m2, e, idx2)

    wa = 1.0 / (1.0 + jnp.exp(m2 - m1))
    wa_ref[...] = wa
    wb_ref[...] = 1.0 - wa

    ci = lax.broadcasted_iota(jnp.int32, (128, 128), 1)
    ri = lax.broadcasted_iota(jnp.int32, (128, 128), 0)
    U = (ri <= ci).astype(jnp.float32)
    ri32 = lax.broadcasted_iota(jnp.int32, (32, 32), 0)
    ci32 = lax.broadcasted_iota(jnp.int32, (32, 32), 1)
    Ls = (ci32 < ri32).astype(jnp.float32)

    dp0 = jnp.zeros((16, 128), jnp.int32)
    dp1 = jnp.zeros((16, 128), jnp.int32)
    tile_iota = lax.broadcasted_iota(jnp.int32, (8, 128), 1)
    eot = jnp.zeros((8, 128), jnp.int32)
    start = jnp.int32(0)
    for e in range(E):
        ind0 = (idx1 == e)
        ind1 = (idx2 == e)
        c0 = lax.dot_general(ind0.astype(jnp.float32), U,
                             (((1,), (0,)), ((), ())),
                             preferred_element_type=jnp.float32)
        c1 = lax.dot_general(ind1.astype(jnp.float32), U,
                             (((1,), (0,)), ((), ())),
                             preferred_element_type=jnp.float32)
        s = jnp.concatenate([c0[:, 127:128], c1[:, 127:128]], axis=0)
        carry = lax.dot_general(Ls, s, (((1,), (0,)), ((), ())),
                                preferred_element_type=jnp.float32)
        inc0 = c0 + carry[0:16]
        inc1 = c1 + carry[16:32]
        cnt = (carry[31, 0] + s[31, 0]).astype(jnp.int32)
        dp0 = jnp.where(ind0, start + inc0.astype(jnp.int32) - 1, dp0)
        dp1 = jnp.where(ind1, start + inc1.astype(jnp.int32) - 1, dp1)
        eot = eot + jnp.where(tile_iota * BT >= start, 1, 0)
        start = start + (cnt + BT - 1) // BT * BT
    dp0_ref[...] = dp0
    dp1_ref[...] = dp1
    eotm = eot - 1
    eot_ref[...] = eotm
    S = (ri == ci - 1).astype(jnp.float32)
    shifted = lax.dot_general(eotm.astype(jnp.float32), S,
                              (((1,), (0,)), ((), ())),
                              preferred_element_type=jnp.float32)
    lane = lax.broadcasted_iota(jnp.int32, (8, 128), 1)
    chb = jnp.logical_or(eotm != shifted.astype(jnp.int32), lane == 0)
    cum = lax.dot_general(chb.astype(jnp.float32), U,
                          (((1,), (0,)), ((), ())),
                          preferred_element_type=jnp.float32)
    flag_ref[...] = chb.astype(jnp.int32)
    slot_ref[...] = (cum.astype(jnp.int32) - 1) & 1


def _route(l3):
    return pl.pallas_call(
        _route_body,
        out_shape=(
            jax.ShapeDtypeStruct((16, 128), jnp.int32),
            jax.ShapeDtypeStruct((16, 128), jnp.int32),
            jax.ShapeDtypeStruct((16, 128), jnp.float32),
            jax.ShapeDtypeStruct((16, 128), jnp.float32),
            jax.ShapeDtypeStruct((8, 128), jnp.int32),
            jax.ShapeDtypeStruct((8, 128), jnp.int32),
            jax.ShapeDtypeStruct((8, 128), jnp.int32),
        ),
    )(l3)


@functools.cache
def _build_dispatch():
    mesh = plsc.VectorSubcoreMesh(core_axis_name="c", subcore_axis_name="s")

    @functools.partial(
        pl.kernel,
        mesh=mesh,
        out_type=jax.ShapeDtypeStruct((P, H // 2), jnp.int32),
        scratch_types=[
            pltpu.VMEM((TPW,), jnp.int32),
            pltpu.VMEM((TPW,), jnp.int32),
            pltpu.VMEM((TPW, H // 2), jnp.int32),
            pltpu.SemaphoreType.DMA,
            pltpu.SemaphoreType.DMA,
        ],
    )
    def dispatch(x_hbm, dp0_hbm, dp1_hbm, xs_hbm, idx0_v, idx1_v, rows_v,
                 sem0, sem1):
        wid = lax.axis_index("s") * 2 + lax.axis_index("c")
        base = wid * TPW
        row, col = wid // 2, (wid % 2) * TPW
        pltpu.sync_copy(dp0_hbm.at[row, pl.ds(col, TPW)], idx0_v)
        pltpu.sync_copy(dp1_hbm.at[row, pl.ds(col, TPW)], idx1_v)
        pltpu.sync_copy(x_hbm.at[pl.ds(base, TPW)], rows_v)
        cp0 = pltpu.async_copy(rows_v, xs_hbm.at[idx0_v], sem0)
        cp1 = pltpu.async_copy(rows_v, xs_hbm.at[idx1_v], sem1)
        cp0.wait()
        cp1.wait()

    return dispatch


_CCH = 32
_NCCH = TPW // _CCH


@functools.cache
def _build_combine():
    mesh = plsc.VectorSubcoreMesh(core_axis_name="c", subcore_axis_name="s")

    @functools.partial(
        pl.kernel,
        mesh=mesh,
        out_type=jax.ShapeDtypeStruct((T, H), jnp.float32),
        scratch_types=[
            pltpu.VMEM((TPW,), jnp.int32),
            pltpu.VMEM((TPW,), jnp.int32),
            pltpu.VMEM((TPW,), jnp.float32),
            pltpu.VMEM((TPW,), jnp.float32),
            pltpu.VMEM((_CCH, H), jnp.float32),
            pltpu.VMEM((_CCH, H), jnp.float32),
            pltpu.SemaphoreType.DMA,
            pltpu.SemaphoreType.DMA,
        ],
    )
    def combine(y_hbm, dp0_hbm, dp1_hbm, wa_hbm, wb_hbm, out_hbm,
                idx0_v, idx1_v, wa_v, wb_v, ya_v, yb_v, sem0, sem1):
        wid = lax.axis_index("s") * 2 + lax.axis_index("c")
        base = wid * TPW
        row, col = wid // 2, (wid % 2) * TPW
        pltpu.sync_copy(dp0_hbm.at[row, pl.ds(col, TPW)], idx0_v)
        pltpu.sync_copy(dp1_hbm.at[row, pl.ds(col, TPW)], idx1_v)
        pltpu.sync_copy(wa_hbm.at[row, pl.ds(col, TPW)], wa_v)
        pltpu.sync_copy(wb_hbm.at[row, pl.ds(col, TPW)], wb_v)
        for c in range(_NCCH):
            cpa = pltpu.async_copy(y_hbm.at[idx0_v.at[pl.ds(c * _CCH, _CCH)]],
                                   ya_v, sem0)
            cpb = pltpu.async_copy(y_hbm.at[idx1_v.at[pl.ds(c * _CCH, _CCH)]],
                                   yb_v, sem1)
            cpa.wait()
            cpb.wait()

            dnums = lax.GatherDimensionNumbers(
                offset_dims=(), collapsed_slice_dims=(0,),
                start_index_map=(0,))
            for g in range(_CCH // 16):
                wva = wa_v[pl.ds(c * _CCH + g * 16, 16)]
                wvb = wb_v[pl.ds(c * _CCH + g * 16, 16)]

                def token_body(t, _, wva=wva, wvb=wvb, g=g):
                    idx = jnp.full((16, 1), t, jnp.int32)
                    wav = lax.gather(wva, idx, dnums, (1,),
                                     mode=lax.GatherScatterMode.PROMISE_IN_BOUNDS)
                    wbv = lax.gather(wvb, idx, dnums, (1,),
                                     mode=lax.GatherScatterMode.PROMISE_IN_BOUNDS)
                    row = g * 16 + t
                    for j in range(H // 16):
                        sl = pl.ds(j * 16, 16)
                        ya_v[row, sl] = wav * ya_v[row, sl] + wbv * yb_v[row, sl]
                    return 0

                lax.fori_loop(0, 16, token_body, 0)
            pltpu.sync_copy(ya_v, out_hbm.at[pl.ds(base + c * _CCH, _CCH)])

    return combine


def _gemm_body(eot_ref, flag_ref, slot_ref, xs_ref, w1_hbm, w3_hbm, w2_hbm,
               y_ref, w1s, w3s, w2s, sem):
    i = pl.program_id(0)

    def issue(k):
        @pl.when(flag_ref[0, k] == 1)
        def _():
            e = eot_ref[0, k]
            s = slot_ref[0, k]
            pltpu.make_async_copy(w1_hbm.at[e], w1s.at[s], sem.at[0, s]).start()
            pltpu.make_async_copy(w3_hbm.at[e], w3s.at[s], sem.at[1, s]).start()
            pltpu.make_async_copy(w2_hbm.at[e], w2s.at[s], sem.at[2, s]).start()

    @pl.when(i == 0)
    def _():
        issue(0)
        issue(1)
        issue(2)

    @pl.when((i > 0) & (i + 2 < NTILES))
    def _():
        issue(i + 2)

    s_i = slot_ref[0, i]

    @pl.when(flag_ref[0, i] == 1)
    def _():
        e = eot_ref[0, i]
        pltpu.make_async_copy(w1_hbm.at[e], w1s.at[s_i], sem.at[0, s_i]).wait()
        pltpu.make_async_copy(w3_hbm.at[e], w3s.at[s_i], sem.at[1, s_i]).wait()
        pltpu.make_async_copy(w2_hbm.at[e], w2s.at[s_i], sem.at[2, s_i]).wait()

    x = xs_ref[...]
    w1 = w1s[s_i].astype(jnp.bfloat16)
    w3 = w3s[s_i].astype(jnp.bfloat16)
    w2 = w2s[s_i].astype(jnp.bfloat16)
    gate = lax.dot_general(x, w1, (((1,), (1,)), ((), ())),
                           preferred_element_type=jnp.float32)
    up = lax.dot_general(x, w3, (((1,), (1,)), ((), ())),
                         preferred_element_type=jnp.float32)
    h = (gate * jax.nn.sigmoid(gate) * up).astype(jnp.bfloat16)
    y_ref[...] = lax.dot_general(h, w2, (((1,), (1,)), ((), ())),
                                 preferred_element_type=jnp.float32)


def _grouped_gemm(xs, w1, w3, w2, eot, flag, slot):
    grid_spec = pltpu.PrefetchScalarGridSpec(
        num_scalar_prefetch=3,
        grid=(NTILES,),
        in_specs=[
            pl.BlockSpec((BT, H), lambda i, eot, flag, slot: (i, 0)),
            pl.BlockSpec(memory_space=pl.ANY),
            pl.BlockSpec(memory_space=pl.ANY),
            pl.BlockSpec(memory_space=pl.ANY),
        ],
        out_specs=pl.BlockSpec((BT, H), lambda i, eot, flag, slot: (i, 0)),
        scratch_shapes=[
            pltpu.VMEM((2, I, H), jnp.float32),
            pltpu.VMEM((2, I, H), jnp.float32),
            pltpu.VMEM((2, H, I), jnp.float32),
            pltpu.SemaphoreType.DMA((3, 2)),
        ],
    )
    return pl.pallas_call(
        _gemm_body,
        grid_spec=grid_spec,
        out_shape=jax.ShapeDtypeStruct((P, H), jnp.float32),
    )(eot, flag, slot, xs, w1, w3, w2)


def kernel(hidden_states, router_logits, w1, w3, w2):
    x = hidden_states.reshape(T, H)

    l3 = router_logits.T.reshape(E, 16, 128)
    dp0g, dp1g, wag, wbg, eotg, flagg, slotg = _route(l3)

    xpk = lax.bitcast_convert_type(
        x.astype(jnp.bfloat16).reshape(T, H // 2, 2), jnp.int32)
    xspk = _build_dispatch()(xpk, dp0g, dp1g)
    xs = lax.bitcast_convert_type(xspk, jnp.bfloat16).reshape(P, H)
    y = _grouped_gemm(xs, w1, w3, w2, eotg, flagg, slotg)
    out = _build_combine()(y, dp0g, dp1g, wag, wbg)
    return out


# speedup vs baseline: 2.0285x; 2.0285x over previous
import functools
import jax
import jax.numpy as jnp
from jax import lax
from jax.experimental import pallas as pl
from jax.experimental.pallas import tpu as pltpu
from jax.experimental.pallas import tpu_sc as plsc

E = 8
K = 2
H = 1024
I = 2048
T = 2048

BT = 256
P = ((K * T + E * (BT - 1)) + BT - 1) // BT * BT
NTILES = P // BT
NEG_INF = -1e30

NW = 32
TPW = T // NW


def _route_body(l3_ref, dp0_ref, dp1_ref, wa_ref, wb_ref, eot_ref,
                flag_ref, slot_ref):
    sl = [l3_ref[e] for e in range(E)]
    m1 = sl[0]
    for e in range(1, E):
        m1 = jnp.maximum(m1, sl[e])
    idx1 = jnp.zeros((16, 128), jnp.int32)
    for e in range(E - 1, -1, -1):
        idx1 = jnp.where(sl[e] == m1, e, idx1)
    m2 = jnp.full((16, 128), NEG_INF, jnp.float32)
    for e in range(E):
        m2 = jnp.maximum(m2, jnp.where(idx1 == e, NEG_INF, sl[e]))
    idx2 = jnp.zeros((16, 128), jnp.int32)
    for e in range(E - 1, -1, -1):
        masked = jnp.where(idx1 == e, NEG_INF, sl[e])
        idx2 = jnp.where(masked == m2, e, idx2)

    wa = 1.0 / (1.0 + jnp.exp(m2 - m1))
    wa_ref[...] = wa
    wb_ref[...] = 1.0 - wa

    ci = lax.broadcasted_iota(jnp.int32, (128, 128), 1)
    ri = lax.broadcasted_iota(jnp.int32, (128, 128), 0)
    U = (ri <= ci).astype(jnp.float32)
    ri32 = lax.broadcasted_iota(jnp.int32, (32, 32), 0)
    ci32 = lax.broadcasted_iota(jnp.int32, (32, 32), 1)
    Ls = (ci32 < ri32).astype(jnp.float32)

    dp0 = jnp.zeros((16, 128), jnp.int32)
    dp1 = jnp.zeros((16, 128), jnp.int32)
    tile_iota = lax.broadcasted_iota(jnp.int32, (8, 128), 1)
    eot = jnp.zeros((8, 128), jnp.int32)
    start = jnp.int32(0)
    for e in range(E):
        ind0 = (idx1 == e)
        ind1 = (idx2 == e)
        c0 = lax.dot_general(ind0.astype(jnp.float32), U,
                             (((1,), (0,)), ((), ())),
                             preferred_element_type=jnp.float32)
        c1 = lax.dot_general(ind1.astype(jnp.float32), U,
                             (((1,), (0,)), ((), ())),
                             preferred_element_type=jnp.float32)
        s = jnp.concatenate([c0[:, 127:128], c1[:, 127:128]], axis=0)
        carry = lax.dot_general(Ls, s, (((1,), (0,)), ((), ())),
                                preferred_element_type=jnp.float32)
        inc0 = c0 + carry[0:16]
        inc1 = c1 + carry[16:32]
        cnt = (carry[31, 0] + s[31, 0]).astype(jnp.int32)
        dp0 = jnp.where(ind0, start + inc0.astype(jnp.int32) - 1, dp0)
        dp1 = jnp.where(ind1, start + inc1.astype(jnp.int32) - 1, dp1)
        eot = eot + jnp.where(tile_iota * BT >= start, 1, 0)
        start = start + (cnt + BT - 1) // BT * BT
    dp0_ref[...] = dp0
    dp1_ref[...] = dp1
    eotm = eot - 1
    eot_ref[...] = eotm
    S = (ri == ci - 1).astype(jnp.float32)
    shifted = lax.dot_general(eotm.astype(jnp.float32), S,
                              (((1,), (0,)), ((), ())),
                              preferred_element_type=jnp.float32)
    lane = lax.broadcasted_iota(jnp.int32, (8, 128), 1)
    chb = jnp.logical_or(eotm != shifted.astype(jnp.int32), lane == 0)
    cum = lax.dot_general(chb.astype(jnp.float32), U,
                          (((1,), (0,)), ((), ())),
                          preferred_element_type=jnp.float32)
    flag_ref[...] = chb.astype(jnp.int32)
    slot_ref[...] = (cum.astype(jnp.int32) - 1) & 1


def _route(l3):
    return pl.pallas_call(
        _route_body,
        out_shape=(
            jax.ShapeDtypeStruct((16, 128), jnp.int32),
            jax.ShapeDtypeStruct((16, 128), jnp.int32),
            jax.ShapeDtypeStruct((16, 128), jnp.float32),
            jax.ShapeDtypeStruct((16, 128), jnp.float32),
            jax.ShapeDtypeStruct((8, 128), jnp.int32),
            jax.ShapeDtypeStruct((8, 128), jnp.int32),
            jax.ShapeDtypeStruct((8, 128), jnp.int32),
        ),
    )(l3)


@functools.cache
def _build_dispatch():
    mesh = plsc.VectorSubcoreMesh(core_axis_name="c", subcore_axis_name="s")

    @functools.partial(
        pl.kernel,
        mesh=mesh,
        out_type=jax.ShapeDtypeStruct((P, H), jnp.float32),
        scratch_types=[
            pltpu.VMEM((TPW,), jnp.int32),
            pltpu.VMEM((TPW,), jnp.int32),
            pltpu.VMEM((TPW, H), jnp.float32),
            pltpu.SemaphoreType.DMA,
            pltpu.SemaphoreType.DMA,
        ],
    )
    def dispatch(x_hbm, dp0_hbm, dp1_hbm, xs_hbm, idx0_v, idx1_v, rows_v,
                 sem0, sem1):
        wid = lax.axis_index("s") * 2 + lax.axis_index("c")
        base = wid * TPW
        row, col = wid // 2, (wid % 2) * TPW
        pltpu.sync_copy(dp0_hbm.at[row, pl.ds(col, TPW)], idx0_v)
        pltpu.sync_copy(dp1_hbm.at[row, pl.ds(col, TPW)], idx1_v)
        pltpu.sync_copy(x_hbm.at[pl.ds(base, TPW)], rows_v)
        cp0 = pltpu.async_copy(rows_v, xs_hbm.at[idx0_v], sem0)
        cp1 = pltpu.async_copy(rows_v, xs_hbm.at[idx1_v], sem1)
        cp0.wait()
        cp1.wait()

    return dispatch


_CCH = 32
_NCCH = TPW // _CCH


@functools.cache
def _build_combine():
    mesh = plsc.VectorSubcoreMesh(core_axis_name="c", subcore_axis_name="s")

    @functools.partial(
        pl.kernel,
        mesh=mesh,
        out_type=jax.ShapeDtypeStruct((T, H), jnp.float32),
        scratch_types=[
            pltpu.VMEM((TPW,), jnp.int32),
            pltpu.VMEM((TPW,), jnp.int32),
            pltpu.VMEM((TPW,), jnp.float32),
            pltpu.VMEM((TPW,), jnp.float32),
            pltpu.VMEM((_CCH, H), jnp.float32),
            pltpu.VMEM((_CCH, H), jnp.float32),
            pltpu.SemaphoreType.DMA,
            pltpu.SemaphoreType.DMA,
        ],
    )
    def combine(y_hbm, dp0_hbm, dp1_hbm, wa_hbm, wb_hbm, out_hbm,
                idx0_v, idx1_v, wa_v, wb_v, ya_v, yb_v, sem0, sem1):
        wid = lax.axis_index("s") * 2 + lax.axis_index("c")
        base = wid * TPW
        row, col = wid // 2, (wid % 2) * TPW
        pltpu.sync_copy(dp0_hbm.at[row, pl.ds(col, TPW)], idx0_v)
        pltpu.sync_copy(dp1_hbm.at[row, pl.ds(col, TPW)], idx1_v)
        pltpu.sync_copy(wa_hbm.at[row, pl.ds(col, TPW)], wa_v)
        pltpu.sync_copy(wb_hbm.at[row, pl.ds(col, TPW)], wb_v)
        for c in range(_NCCH):
            cpa = pltpu.async_copy(y_hbm.at[idx0_v.at[pl.ds(c * _CCH, _CCH)]],
                                   ya_v, sem0)
            cpb = pltpu.async_copy(y_hbm.at[idx1_v.at[pl.ds(c * _CCH, _CCH)]],
                                   yb_v, sem1)
            cpa.wait()
            cpb.wait()

            dnums = lax.GatherDimensionNumbers(
                offset_dims=(), collapsed_slice_dims=(0,),
                start_index_map=(0,))
            for g in range(_CCH // 16):
                wva = wa_v[pl.ds(c * _CCH + g * 16, 16)]
                wvb = wb_v[pl.ds(c * _CCH + g * 16, 16)]

                def token_body(t, _, wva=wva, wvb=wvb, g=g):
                    idx = jnp.full((16, 1), t, jnp.int32)
                    wav = lax.gather(wva, idx, dnums, (1,),
                                     mode=lax.GatherScatterMode.PROMISE_IN_BOUNDS)
                    wbv = lax.gather(wvb, idx, dnums, (1,),
                                     mode=lax.GatherScatterMode.PROMISE_IN_BOUNDS)
                    row = g * 16 + t
                    for j in range(H // 16):
                        sl = pl.ds(j * 16, 16)
                        ya_v[row, sl] = wav * ya_v[row, sl] + wbv * yb_v[row, sl]
                    return 0

                lax.fori_loop(0, 16, token_body, 0)
            pltpu.sync_copy(ya_v, out_hbm.at[pl.ds(base + c * _CCH, _CCH)])

    return combine


def _gemm_body(eot_ref, flag_ref, slot_ref, xs_ref, w1_hbm, w3_hbm, w2_hbm,
               y_ref, w1s, w3s, w2s, sem):
    i = pl.program_id(0)

    def issue(k):
        @pl.when(flag_ref[0, k] == 1)
        def _():
            e = eot_ref[0, k]
            s = slot_ref[0, k]
            pltpu.make_async_copy(w1_hbm.at[e], w1s.at[s], sem.at[0, s]).start()
            pltpu.make_async_copy(w3_hbm.at[e], w3s.at[s], sem.at[1, s]).start()
            pltpu.make_async_copy(w2_hbm.at[e], w2s.at[s], sem.at[2, s]).start()

    @pl.when(i == 0)
    def _():
        issue(0)
        issue(1)
        issue(2)

    @pl.when((i > 0) & (i + 2 < NTILES))
    def _():
        issue(i + 2)

    s_i = slot_ref[0, i]

    @pl.when(flag_ref[0, i] == 1)
    def _():
        e = eot_ref[0, i]
        pltpu.make_async_copy(w1_hbm.at[e], w1s.at[s_i], sem.at[0, s_i]).wait()
        pltpu.make_async_copy(w3_hbm.at[e], w3s.at[s_i], sem.at[1, s_i]).wait()
        pltpu.make_async_copy(w2_hbm.at[e], w2s.at[s_i], sem.at[2, s_i]).wait()

    x = xs_ref[...].astype(jnp.bfloat16)
    w1 = w1s[s_i].astype(jnp.bfloat16)
    w3 = w3s[s_i].astype(jnp.bfloat16)
    w2 = w2s[s_i].astype(jnp.bfloat16)
    gate = lax.dot_general(x, w1, (((1,), (1,)), ((), ())),
                           preferred_element_type=jnp.float32)
    up = lax.dot_general(x, w3, (((1,), (1,)), ((), ())),
                         preferred_element_type=jnp.float32)
    h = (gate * jax.nn.sigmoid(gate) * up).astype(jnp.bfloat16)
    y_ref[...] = lax.dot_general(h, w2, (((1,), (1,)), ((), ())),
                                 preferred_element_type=jnp.float32)


def _grouped_gemm(xs, w1, w3, w2, eot, flag, slot):
    grid_spec = pltpu.PrefetchScalarGridSpec(
        num_scalar_prefetch=3,
        grid=(NTILES,),
        in_specs=[
            pl.BlockSpec((BT, H), lambda i, eot, flag, slot: (i, 0)),
            pl.BlockSpec(memory_space=pl.ANY),
            pl.BlockSpec(memory_space=pl.ANY),
            pl.BlockSpec(memory_space=pl.ANY),
        ],
        out_specs=pl.BlockSpec((BT, H), lambda i, eot, flag, slot: (i, 0)),
        scratch_shapes=[
            pltpu.VMEM((2, I, H), jnp.float32),
            pltpu.VMEM((2, I, H), jnp.float32),
            pltpu.VMEM((2, H, I), jnp.float32),
            pltpu.SemaphoreType.DMA((3, 2)),
        ],
    )
    return pl.pallas_call(
        _gemm_body,
        grid_spec=grid_spec,
        out_shape=jax.ShapeDtypeStruct((P, H), jnp.float32),
    )(eot, flag, slot, xs, w1, w3, w2)


def kernel(hidden_states, router_logits, w1, w3, w2):
    x = hidden_states.reshape(T, H)

    l3 = router_logits.T.reshape(E, 16, 128)
    dp0g, dp1g, wag, wbg, eotg, flagg, slotg = _route(l3)

    xs = _build_dispatch()(x, dp0g, dp1g)
    y = _grouped_gemm(xs, w1, w3, w2, eotg, flagg, slotg)
    out = _build_combine()(y, dp0g, dp1g, wag, wbg)
    return out
